# baseline (speedup 1.0000x reference)
import functools

import jax
import jax.numpy as jnp
from jax import lax
from jax.experimental import pallas as pl
from jax.experimental.pallas import tpu as pltpu
from jax.experimental.pallas import tpu_sc as plsc

B = 16384
L = 200
EMB = 128
VPAD = 16

NUM_WORKERS = 32
ROWS_PER_W = B // NUM_WORKERS
GROUPS = ROWS_PER_W // 16


def _sc_histogram_t(sk_t, lengths):
    mesh = plsc.VectorSubcoreMesh(core_axis_name="c", subcore_axis_name="s")

    @functools.partial(
        pl.kernel,
        mesh=mesh,
        out_type=jax.ShapeDtypeStruct((VPAD, B), jnp.float32),
        compiler_params=pltpu.CompilerParams(
            use_tc_tiling_on_sc=True, needs_layout_passes=False
        ),
        scratch_types=[
            pltpu.VMEM((L, ROWS_PER_W), jnp.int32),
            pltpu.VMEM((ROWS_PER_W,), jnp.int32),
            pltpu.VMEM((VPAD * ROWS_PER_W,), jnp.float32),
            pltpu.VMEM((VPAD, ROWS_PER_W), jnp.float32),
        ],
    )
    def hist(skt_hbm, len_hbm, counts_hbm, slab_v, lens_v, acc_v, stage_v):
        wid = lax.axis_index("s") * 2 + lax.axis_index("c")
        base = wid * ROWS_PER_W
        pltpu.sync_copy(skt_hbm.at[:, pl.ds(base, ROWS_PER_W)], slab_v)
        pltpu.sync_copy(len_hbm.at[pl.ds(base, ROWS_PER_W)], lens_v)

        iota = lax.iota(jnp.int32, 16)
        ones = jnp.ones((16,), jnp.float32)
        zeros = jnp.zeros((16,), jnp.float32)

        @plsc.parallel_loop(0, VPAD * ROWS_PER_W, step=16, unroll=8)
        def _zero(i):
            acc_v[pl.ds(i, 16)] = zeros

        def group_body(g, carry):
            rows = g * 16 + iota
            lens16 = lens_v[pl.ds(g * 16, 16)]

            @plsc.parallel_loop(0, L, unroll=25)
            def _pos(l):
                vals = slab_v[l, pl.ds(g * 16, 16)]
                mask = l < lens16
                plsc.addupdate_scatter(
                    acc_v, [vals * ROWS_PER_W + rows], ones, mask=mask
                )

            return carry

        lax.fori_loop(0, GROUPS, group_body, 0)

        for v in range(VPAD):
            @plsc.parallel_loop(0, ROWS_PER_W, step=16, unroll=8)
            def _stage(i, _v=v):
                stage_v[_v, pl.ds(i, 16)] = acc_v[pl.ds(_v * ROWS_PER_W + i, 16)]

        pltpu.sync_copy(stage_v, counts_hbm.at[:, pl.ds(base, ROWS_PER_W)])

    return hist(sk_t, lengths)


def _tc_matmul(counts_t, table_pad):
    BM = 4096

    def mm(counts_ref, table_ref, out_ref):
        out_ref[...] = lax.dot_general(
            counts_ref[...],
            table_ref[...],
            (((0,), (0,)), ((), ())),
            preferred_element_type=jnp.float32,
        )

    return pl.pallas_call(
        mm,
        grid=(B // BM,),
        in_specs=[
            pl.BlockSpec((VPAD, BM), lambda i: (0, i)),
            pl.BlockSpec((VPAD, EMB), lambda i: (0, 0)),
        ],
        out_specs=pl.BlockSpec((BM, EMB), lambda i: (i, 0)),
        out_shape=jax.ShapeDtypeStruct((B, EMB), jnp.float32),
    )(counts_t, table_pad)


def kernel(sketchs, sketch_lengths, table):
    sk_t = jnp.transpose(jnp.asarray(sketchs, jnp.int32))
    lengths = jnp.asarray(sketch_lengths, jnp.int32)
    table_pad = jnp.zeros((VPAD, EMB), jnp.float32).at[:10, :].set(table)
    counts_t = _sc_histogram_t(sk_t, lengths)
    return _tc_matmul(counts_t, table_pad)


# speedup vs baseline: 1.1960x; 1.1960x over previous
import functools

import jax
import jax.numpy as jnp
from jax import lax
from jax.experimental import pallas as pl
from jax.experimental.pallas import tpu as pltpu
from jax.experimental.pallas import tpu_sc as plsc

B = 16384
L = 200
EMB = 128
VPAD = 16

NUM_WORKERS = 32
ROWS_PER_W = B // NUM_WORKERS
GROUPS = ROWS_PER_W // 16


def _sc_histogram_t(sk_t, lengths):
    mesh = plsc.VectorSubcoreMesh(core_axis_name="c", subcore_axis_name="s")

    @functools.partial(
        pl.kernel,
        mesh=mesh,
        out_type=jax.ShapeDtypeStruct((VPAD, B), jnp.float32),
        compiler_params=pltpu.CompilerParams(
            use_tc_tiling_on_sc=True, needs_layout_passes=False
        ),
        scratch_types=[
            pltpu.VMEM((L, ROWS_PER_W), jnp.int32),
            pltpu.VMEM((ROWS_PER_W,), jnp.int32),
            pltpu.VMEM((VPAD * ROWS_PER_W,), jnp.float32),
            pltpu.VMEM((VPAD, ROWS_PER_W), jnp.float32),
        ],
    )
    def hist(skt_hbm, len_hbm, counts_hbm, slab_v, lens_v, acc_v, stage_v):
        wid = lax.axis_index("s") * 2 + lax.axis_index("c")
        base = wid * ROWS_PER_W
        pltpu.sync_copy(skt_hbm.at[:, pl.ds(base, ROWS_PER_W)], slab_v)
        pltpu.sync_copy(len_hbm.at[pl.ds(base, ROWS_PER_W)], lens_v)

        iota = lax.iota(jnp.int32, 16)
        ones = jnp.ones((16,), jnp.float32)
        zeros = jnp.zeros((16,), jnp.float32)

        @plsc.parallel_loop(0, VPAD * ROWS_PER_W, step=16, unroll=8)
        def _zero(i):
            acc_v[pl.ds(i, 16)] = zeros

        def group_body(g, carry):
            rows = g * 16 + iota
            lens16 = lens_v[pl.ds(g * 16, 16)]

            @plsc.parallel_loop(0, L, unroll=8)
            def _pos(l):
                vals = slab_v[l, pl.ds(g * 16, 16)]
                mask = l < lens16
                plsc.addupdate_scatter(
                    acc_v, [vals * ROWS_PER_W + rows], ones, mask=mask
                )

            return carry

        lax.fori_loop(0, GROUPS, group_body, 0)

        for v in range(VPAD):
            @plsc.parallel_loop(0, ROWS_PER_W, step=16, unroll=8)
            def _stage(i, _v=v):
                stage_v[_v, pl.ds(i, 16)] = acc_v[pl.ds(_v * ROWS_PER_W + i, 16)]

        pltpu.sync_copy(stage_v, counts_hbm.at[:, pl.ds(base, ROWS_PER_W)])

    return hist(sk_t, lengths)


def _tc_matmul(counts_t, table_pad):
    BM = 4096

    def mm(counts_ref, table_ref, out_ref):
        out_ref[...] = lax.dot_general(
            counts_ref[...],
            table_ref[...],
            (((0,), (0,)), ((), ())),
            preferred_element_type=jnp.float32,
        )

    return pl.pallas_call(
        mm,
        grid=(B // BM,),
        in_specs=[
            pl.BlockSpec((VPAD, BM), lambda i: (0, i)),
            pl.BlockSpec((VPAD, EMB), lambda i: (0, 0)),
        ],
        out_specs=pl.BlockSpec((BM, EMB), lambda i: (i, 0)),
        out_shape=jax.ShapeDtypeStruct((B, EMB), jnp.float32),
    )(counts_t, table_pad)


def kernel(sketchs, sketch_lengths, table):
    sk_t = jnp.transpose(jnp.asarray(sketchs, jnp.int32))
    lengths = jnp.asarray(sketch_lengths, jnp.int32)
    table_pad = jnp.zeros((VPAD, EMB), jnp.float32).at[:10, :].set(table)
    counts_t = _sc_histogram_t(sk_t, lengths)
    return _tc_matmul(counts_t, table_pad)
